# Initial kernel scaffold; baseline (speedup 1.0000x reference)
import functools

import jax
import jax.numpy as jnp
from jax import lax
from jax.experimental import pallas as pl
from jax.experimental.pallas import tpu as pltpu
from jax.experimental.pallas import tpu_sc as plsc

NC = 2
NS = 16
NW = NC * NS
K = 125


def _sc_mesh():
    return plsc.VectorSubcoreMesh(core_axis_name="c", subcore_axis_name="s")


def _make_deg(N, CH):
    RPS = N // NS

    @functools.partial(
        pl.kernel,
        out_type=jax.ShapeDtypeStruct((NC, 2, N, 16), jnp.float32),
        mesh=_sc_mesh(),
        scratch_types=[
            pltpu.VMEM_SHARED((N, 16), jnp.float32),
            pltpu.VMEM_SHARED((N, 16), jnp.float32),
            pltpu.VMEM((CH, K), jnp.int32),
            pltpu.VMEM((CH, K), jnp.int32),
            pltpu.VMEM((K, 16), jnp.float32),
        ],
    )
    def deg(src_hbm, dst_hbm, z_hbm, ones_hbm, out_hbm,
            cs_sh, cd_sh, isv, idv, ones_v):
        cid = lax.axis_index("c")
        sid = lax.axis_index("s")
        wid = cid * NS + sid
        r0 = sid * RPS
        pltpu.sync_copy(z_hbm.at[pl.ds(r0, RPS)], cs_sh.at[pl.ds(r0, RPS)])
        pltpu.sync_copy(z_hbm.at[pl.ds(r0, RPS)], cd_sh.at[pl.ds(r0, RPS)])
        pltpu.sync_copy(ones_hbm, ones_v)
        pltpu.sync_copy(src_hbm.at[wid], isv)
        pltpu.sync_copy(dst_hbm.at[wid], idv)
        plsc.subcore_barrier()

        @pl.loop(0, CH)
        def _(t):
            pltpu.sync_copy(ones_v, cs_sh.at[isv.at[t]], add=True)
            pltpu.sync_copy(ones_v, cd_sh.at[idv.at[t]], add=True)

        plsc.subcore_barrier()
        pltpu.sync_copy(cs_sh.at[pl.ds(r0, RPS)],
                        out_hbm.at[cid, 0, pl.ds(r0, RPS)])
        pltpu.sync_copy(cd_sh.at[pl.ds(r0, RPS)],
                        out_hbm.at[cid, 1, pl.ds(r0, RPS)])

    return deg


def _make_agg(N, D, CH):
    RPS = N // NS

    @functools.partial(
        pl.kernel,
        out_type=jax.ShapeDtypeStruct((NC, N, D), jnp.float32),
        mesh=_sc_mesh(),
        scratch_types=[
            pltpu.VMEM_SHARED((N, D), jnp.float32),
            pltpu.VMEM((CH, K), jnp.int32),
            pltpu.VMEM((CH, K), jnp.int32),
            pltpu.VMEM((K, D), jnp.float32),
        ],
    )
    def agg(hw_hbm, src_hbm, dst_hbm, z_hbm, out_hbm,
            acc_sh, isv, idv, rows_v):
        cid = lax.axis_index("c")
        sid = lax.axis_index("s")
        wid = cid * NS + sid
        r0 = sid * RPS
        pltpu.sync_copy(z_hbm.at[pl.ds(r0, RPS)], acc_sh.at[pl.ds(r0, RPS)])
        pltpu.sync_copy(src_hbm.at[wid], isv)
        pltpu.sync_copy(dst_hbm.at[wid], idv)
        plsc.subcore_barrier()

        @pl.loop(0, CH)
        def _(t):
            pltpu.sync_copy(hw_hbm.at[isv.at[t]], rows_v)
            pltpu.sync_copy(rows_v, acc_sh.at[idv.at[t]], add=True)

        plsc.subcore_barrier()
        pltpu.sync_copy(acc_sh.at[pl.ds(r0, RPS)],
                        out_hbm.at[cid, pl.ds(r0, RPS)])

    return agg


def _dcol(c_ref, which):
    deg = c_ref[0, which] + c_ref[1, which]
    d = lax.rsqrt(jnp.maximum(deg, 1.0))
    return d[:, 0:1]


def _stage0_body(f_ref, c_ref, w_ref, o_ref):
    h = f_ref[...] * _dcol(c_ref, 0)
    o_ref[...] = lax.dot(h, w_ref[...], precision=lax.Precision.HIGHEST)


def _stage_mid_body(a_ref, c_ref, b_ref, w_ref, o_ref):
    a = a_ref[0] + a_ref[1]
    h = jnp.maximum(a * _dcol(c_ref, 1) + b_ref[0:1, :], 0.0)
    o_ref[...] = lax.dot(h * _dcol(c_ref, 0), w_ref[...],
                         precision=lax.Precision.HIGHEST)


def _stage_last_body(a_ref, c_ref, b_ref, o_ref):
    a = a_ref[0] + a_ref[1]
    o_ref[...] = jnp.maximum(a * _dcol(c_ref, 1) + b_ref[0:1, :], 0.0)


def _tc_call(body, N, D, BR, in_specs):
    return pl.pallas_call(
        body,
        grid=(N // BR,),
        in_specs=in_specs,
        out_specs=pl.BlockSpec((BR, D), lambda i: (i, 0)),
        out_shape=jax.ShapeDtypeStruct((N, D), jnp.float32),
    )


def kernel(features, edge_index, W1, b1, W2, b2, W3, b3):
    N, D = features.shape
    E = edge_index.shape[1]
    CH = E // (NW * K)
    BR = 1000

    src3 = edge_index[0].reshape(NW, CH, K)
    dst3 = edge_index[1].reshape(NW, CH, K)
    z16 = jnp.zeros((N, 16), jnp.float32)
    zND = jnp.zeros((N, D), jnp.float32)
    ones = jnp.ones((K, 16), jnp.float32)
    b1b = jnp.broadcast_to(b1.reshape(1, D), (8, D))
    b2b = jnp.broadcast_to(b2.reshape(1, D), (8, D))
    b3b = jnp.broadcast_to(b3.reshape(1, D), (8, D))

    cnt = _make_deg(N, CH)(src3, dst3, z16, ones)
    agg_fn = _make_agg(N, D, CH)

    spec_c = pl.BlockSpec((NC, 2, BR, 16), lambda i: (0, 0, i, 0))
    spec_a = pl.BlockSpec((NC, BR, D), lambda i: (0, i, 0))
    spec_w = pl.BlockSpec((D, D), lambda i: (0, 0))
    spec_b = pl.BlockSpec((8, D), lambda i: (0, 0))
    spec_f = pl.BlockSpec((BR, D), lambda i: (i, 0))

    hw = _tc_call(_stage0_body, N, D, BR, [spec_f, spec_c, spec_w])(
        features, cnt, W1)
    for W, bb in ((W2, b1b), (W3, b2b)):
        a = agg_fn(hw, src3, dst3, zND)
        hw = _tc_call(_stage_mid_body, N, D, BR,
                      [spec_a, spec_c, spec_b, spec_w])(a, cnt, bb, W)
    a = agg_fn(hw, src3, dst3, zND)
    out = _tc_call(_stage_last_body, N, D, BR,
                   [spec_a, spec_c, spec_b])(a, cnt, b3b)
    return out


# speedup vs baseline: 6.6056x; 6.6056x over previous
import functools

import jax
import jax.numpy as jnp
from jax import lax
from jax.experimental import pallas as pl
from jax.experimental.pallas import tpu as pltpu
from jax.experimental.pallas import tpu_sc as plsc

NC = 2
NS = 16
NW = NC * NS
K = 125


def _sc_mesh():
    return plsc.VectorSubcoreMesh(core_axis_name="c", subcore_axis_name="s",
                                  num_cores=NC, num_subcores=NS)


def _stripe(sid, N):
    rps8 = (N // NS) // 8 * 8
    tail = N - NS * rps8
    return sid * rps8, rps8, NS * rps8, tail


def _make_deg(N, D, CH):
    @functools.partial(
        pl.kernel,
        out_type=jax.ShapeDtypeStruct((NC, N, D), jnp.float32),
        mesh=_sc_mesh(),
        scratch_types=[
            pltpu.VMEM_SHARED((N, D), jnp.float32),
            pltpu.VMEM((2 * CH, K), jnp.int32),
            pltpu.VMEM((K, D), jnp.float32),
        ],
    )
    def deg(src_hbm, dst_hbm, z_hbm, ones_s_hbm, ones_d_hbm, out_hbm,
            c_sh, iv, ones_v):
        cid = lax.axis_index("c")
        sid = lax.axis_index("s")
        wid = cid * NS + sid
        r0, rn, t0, tn = _stripe(sid, N)
        pltpu.sync_copy(z_hbm.at[pl.ds(r0, rn)], c_sh.at[pl.ds(r0, rn)])

        @pl.when(jnp.logical_and(sid == 0, tn > 0))
        def _():
            pltpu.sync_copy(z_hbm.at[pl.ds(t0, tn)], c_sh.at[pl.ds(t0, tn)])

        pltpu.sync_copy(ones_s_hbm, ones_v)
        pltpu.sync_copy(src_hbm.at[wid], iv.at[pl.ds(0, CH)])
        pltpu.sync_copy(dst_hbm.at[wid], iv.at[pl.ds(CH, CH)])
        plsc.subcore_barrier()

        @pl.loop(0, 2 * CH)
        def _(t):
            @pl.when(t == CH)
            def _():
                pltpu.sync_copy(ones_d_hbm, ones_v)

            pltpu.sync_copy(ones_v, c_sh.at[iv.at[t]], add=True)

        plsc.subcore_barrier()
        pltpu.sync_copy(c_sh.at[pl.ds(r0, rn)],
                        out_hbm.at[cid, pl.ds(r0, rn)])

        @pl.when(jnp.logical_and(sid == 0, tn > 0))
        def _():
            pltpu.sync_copy(c_sh.at[pl.ds(t0, tn)],
                            out_hbm.at[cid, pl.ds(t0, tn)])

    return deg


def _make_agg(N, D, CH):
    @functools.partial(
        pl.kernel,
        out_type=jax.ShapeDtypeStruct((NC, N, D), jnp.float32),
        mesh=_sc_mesh(),
        scratch_types=[
            pltpu.VMEM_SHARED((N, D), jnp.float32),
            pltpu.VMEM((CH, K), jnp.int32),
            pltpu.VMEM((CH, K), jnp.int32),
            pltpu.VMEM((K, D), jnp.float32),
        ],
    )
    def agg(hw_hbm, src_hbm, dst_hbm, z_hbm, out_hbm,
            acc_sh, isv, idv, rows_v):
        cid = lax.axis_index("c")
        sid = lax.axis_index("s")
        wid = cid * NS + sid
        r0, rn, t0, tn = _stripe(sid, N)
        pltpu.sync_copy(z_hbm.at[pl.ds(r0, rn)], acc_sh.at[pl.ds(r0, rn)])

        @pl.when(jnp.logical_and(sid == 0, tn > 0))
        def _():
            pltpu.sync_copy(z_hbm.at[pl.ds(t0, tn)], acc_sh.at[pl.ds(t0, tn)])

        pltpu.sync_copy(src_hbm.at[wid], isv)
        pltpu.sync_copy(dst_hbm.at[wid], idv)
        plsc.subcore_barrier()

        @pl.loop(0, CH)
        def _(t):
            pltpu.sync_copy(hw_hbm.at[isv.at[t]], rows_v)
            pltpu.sync_copy(rows_v, acc_sh.at[idv.at[t]], add=True)

        plsc.subcore_barrier()
        pltpu.sync_copy(acc_sh.at[pl.ds(r0, rn)],
                        out_hbm.at[cid, pl.ds(r0, rn)])

        @pl.when(jnp.logical_and(sid == 0, tn > 0))
        def _():
            pltpu.sync_copy(acc_sh.at[pl.ds(t0, tn)],
                            out_hbm.at[cid, pl.ds(t0, tn)])

    return agg


def _dcol(c_ref, col):
    deg = c_ref[0] + c_ref[1]
    d = lax.rsqrt(jnp.maximum(deg, 1.0))
    return d[:, col:col + 1]


def _dout(c_ref):
    return _dcol(c_ref, 0)


def _din(c_ref):
    return _dcol(c_ref, c_ref.shape[-1] // 2)


def _stage0_body(f_ref, c_ref, w_ref, o_ref):
    h = f_ref[...] * _dout(c_ref)
    o_ref[...] = lax.dot(h, w_ref[...], precision=lax.Precision.HIGHEST)


def _stage_mid_body(a_ref, c_ref, b_ref, w_ref, o_ref):
    a = a_ref[0] + a_ref[1]
    h = jnp.maximum(a * _din(c_ref) + b_ref[0:1, :], 0.0)
    o_ref[...] = lax.dot(h * _dout(c_ref), w_ref[...],
                         precision=lax.Precision.HIGHEST)


def _stage_last_body(a_ref, c_ref, b_ref, o_ref):
    a = a_ref[0] + a_ref[1]
    o_ref[...] = jnp.maximum(a * _din(c_ref) + b_ref[0:1, :], 0.0)


def _tc_call(body, N, D, BR, in_specs):
    return pl.pallas_call(
        body,
        grid=(N // BR,),
        in_specs=in_specs,
        out_specs=pl.BlockSpec((BR, D), lambda i: (i, 0)),
        out_shape=jax.ShapeDtypeStruct((N, D), jnp.float32),
    )


def kernel(features, edge_index, W1, b1, W2, b2, W3, b3):
    N, D = features.shape
    E = edge_index.shape[1]
    CH = E // (NW * K)
    BR = 1000

    src3 = edge_index[0].reshape(NW, CH, K)
    dst3 = edge_index[1].reshape(NW, CH, K)
    zND = jnp.zeros((N, D), jnp.float32)
    lane = jnp.arange(D) < (D // 2)
    ones_s = jnp.broadcast_to(jnp.where(lane, 1.0, 0.0), (K, D))
    ones_s = ones_s.astype(jnp.float32)
    ones_d = jnp.broadcast_to(jnp.where(lane, 0.0, 1.0), (K, D))
    ones_d = ones_d.astype(jnp.float32)
    b1b = jnp.broadcast_to(b1.reshape(1, D), (8, D))
    b2b = jnp.broadcast_to(b2.reshape(1, D), (8, D))
    b3b = jnp.broadcast_to(b3.reshape(1, D), (8, D))

    cnt = _make_deg(N, D, CH)(src3, dst3, zND, ones_s, ones_d)
    agg_fn = _make_agg(N, D, CH)

    spec_c = pl.BlockSpec((NC, BR, D), lambda i: (0, i, 0))
    spec_a = pl.BlockSpec((NC, BR, D), lambda i: (0, i, 0))
    spec_w = pl.BlockSpec((D, D), lambda i: (0, 0))
    spec_b = pl.BlockSpec((8, D), lambda i: (0, 0))
    spec_f = pl.BlockSpec((BR, D), lambda i: (i, 0))

    hw = _tc_call(_stage0_body, N, D, BR, [spec_f, spec_c, spec_w])(
        features, cnt, W1)
    for W, bb in ((W2, b1b), (W3, b2b)):
        a = agg_fn(hw, src3, dst3, zND)
        hw = _tc_call(_stage_mid_body, N, D, BR,
                      [spec_a, spec_c, spec_b, spec_w])(a, cnt, bb, W)
    a = agg_fn(hw, src3, dst3, zND)
    out = _tc_call(_stage_last_body, N, D, BR,
                   [spec_a, spec_c, spec_b])(a, cnt, b3b)
    return out


# speedup vs baseline: 7.6188x; 1.1534x over previous
import dataclasses
import functools

import jax
import jax.numpy as jnp
from jax import lax
from jax.experimental import pallas as pl
from jax.experimental.pallas import tpu as pltpu
from jax.experimental.pallas import tpu_sc as plsc

NC = 2
NS = 16
NW = NC * NS
KA = 125


def _sc_mesh():
    return plsc.VectorSubcoreMesh(core_axis_name="c", subcore_axis_name="s",
                                  num_cores=NC, num_subcores=NS)


def _stripe(sid, N):
    rps8 = (N // NS) // 8 * 8
    tail = N - NS * rps8
    return sid * rps8, rps8, NS * rps8, tail


def _make_deg(N, D, EPW):
    R, G = EPW // 80, 80 // 16

    @functools.partial(
        pl.kernel,
        out_type=(jax.ShapeDtypeStruct((NW, 1, N), jnp.float32),
                  jax.ShapeDtypeStruct((NW, 1, N), jnp.float32)),
        mesh=_sc_mesh(),
        scratch_types=[
            pltpu.VMEM((N,), jnp.float32),
            pltpu.VMEM((N,), jnp.float32),
            pltpu.VMEM((R, 80), jnp.int32),
            pltpu.VMEM((R, 80), jnp.int32),
        ],
        compiler_params=dataclasses.replace(pltpu.CompilerParams(),
                                            needs_layout_passes=False),
    )
    def deg(src_hbm, dst_hbm, z_hbm, outs_hbm, outd_hbm,
            hs, hd, isv, idv):
        cid = lax.axis_index("c")
        sid = lax.axis_index("s")
        wid = cid * NS + sid
        pltpu.sync_copy(z_hbm, hs)
        pltpu.sync_copy(z_hbm, hd)
        pltpu.sync_copy(src_hbm.at[wid], isv)
        pltpu.sync_copy(dst_hbm.at[wid], idv)
        ones16 = jnp.full((16,), 1.0, jnp.float32)

        @pl.loop(0, R)
        def _(r):
            for j in range(G):
                vs = isv[r, pl.ds(j * 16, 16)]
                plsc.addupdate_scatter(hs, [vs], ones16)
                vd = idv[r, pl.ds(j * 16, 16)]
                plsc.addupdate_scatter(hd, [vd], ones16)

        pltpu.sync_copy(hs, outs_hbm.at[wid, 0])
        pltpu.sync_copy(hd, outd_hbm.at[wid, 0])

    return deg


def _dred_body(hs_ref, hd_ref, ss_ref, sd_ref, o_ref):
    dn = (((0,), (0,)), ((), ()))
    acc = lax.dot_general(hs_ref[...], ss_ref[...], dn,
                          precision=lax.Precision.HIGHEST)
    acc = acc + lax.dot_general(hd_ref[...], sd_ref[...], dn,
                                precision=lax.Precision.HIGHEST)
    o_ref[...] = lax.rsqrt(jnp.maximum(acc, 1.0))


def _make_agg(N, D, CH, K):
    @functools.partial(
        pl.kernel,
        out_type=jax.ShapeDtypeStruct((NC, N, D), jnp.float32),
        mesh=_sc_mesh(),
        scratch_types=[
            pltpu.VMEM_SHARED((N, D), jnp.float32),
            pltpu.VMEM((CH, K), jnp.int32),
            pltpu.VMEM((CH, K), jnp.int32),
            pltpu.VMEM((K, D), jnp.float32),
        ],
    )
    def agg(hw_hbm, src_hbm, dst_hbm, z_hbm, out_hbm,
            acc_sh, isv, idv, rows):
        cid = lax.axis_index("c")
        sid = lax.axis_index("s")
        wid = cid * NS + sid
        r0, rn, t0, tn = _stripe(sid, N)
        pltpu.sync_copy(z_hbm.at[pl.ds(r0, rn)], acc_sh.at[pl.ds(r0, rn)])

        @pl.when(jnp.logical_and(sid == 0, tn > 0))
        def _():
            pltpu.sync_copy(z_hbm.at[pl.ds(t0, tn)], acc_sh.at[pl.ds(t0, tn)])

        pltpu.sync_copy(src_hbm.at[wid], isv)
        pltpu.sync_copy(dst_hbm.at[wid], idv)
        plsc.subcore_barrier()

        @pl.loop(0, CH)
        def _(t):
            pltpu.sync_copy(hw_hbm.at[isv.at[t]], rows)
            pltpu.sync_copy(rows, acc_sh.at[idv.at[t]], add=True)

        plsc.subcore_barrier()
        pltpu.sync_copy(acc_sh.at[pl.ds(r0, rn)],
                        out_hbm.at[cid, pl.ds(r0, rn)])

        @pl.when(jnp.logical_and(sid == 0, tn > 0))
        def _():
            pltpu.sync_copy(acc_sh.at[pl.ds(t0, tn)],
                            out_hbm.at[cid, pl.ds(t0, tn)])

    return agg


def _dcol(c_ref, col):
    return c_ref[:, col:col + 1]


def _dout(c_ref):
    return _dcol(c_ref, 0)


def _din(c_ref):
    return _dcol(c_ref, c_ref.shape[-1] // 2)


def _stage0_body(f_ref, c_ref, w_ref, o_ref):
    h = f_ref[...] * _dout(c_ref)
    o_ref[...] = lax.dot(h, w_ref[...], precision=lax.Precision.HIGHEST)


def _stage_mid_body(a_ref, c_ref, b_ref, w_ref, o_ref):
    a = a_ref[0] + a_ref[1]
    h = jnp.maximum(a * _din(c_ref) + b_ref[0:1, :], 0.0)
    o_ref[...] = lax.dot(h * _dout(c_ref), w_ref[...],
                         precision=lax.Precision.HIGHEST)


def _stage_last_body(a_ref, c_ref, b_ref, o_ref):
    a = a_ref[0] + a_ref[1]
    o_ref[...] = jnp.maximum(a * _din(c_ref) + b_ref[0:1, :], 0.0)


def _tc_call(body, N, D, BR, in_specs):
    return pl.pallas_call(
        body,
        grid=(N // BR,),
        in_specs=in_specs,
        out_specs=pl.BlockSpec((BR, D), lambda i: (i, 0)),
        out_shape=jax.ShapeDtypeStruct((N, D), jnp.float32),
    )


def kernel(features, edge_index, W1, b1, W2, b2, W3, b3):
    N, D = features.shape
    E = edge_index.shape[1]
    CHA = E // (NW * KA)
    BR = 1000

    EPW = E // NW
    srcd = edge_index[0].reshape(NW, EPW // 80, 80)
    dstd = edge_index[1].reshape(NW, EPW // 80, 80)
    srca = edge_index[0].reshape(NW, CHA, KA)
    dsta = edge_index[1].reshape(NW, CHA, KA)
    zND = jnp.zeros((N, D), jnp.float32)
    zN = jnp.zeros((N,), jnp.float32)
    lane = jnp.arange(D) < (D // 2)
    sel_s = jnp.broadcast_to(jnp.where(lane, 1.0, 0.0), (NW, D))
    sel_s = sel_s.astype(jnp.float32)
    sel_d = jnp.broadcast_to(jnp.where(lane, 0.0, 1.0), (NW, D))
    sel_d = sel_d.astype(jnp.float32)
    b1b = jnp.broadcast_to(b1.reshape(1, D), (8, D))
    b2b = jnp.broadcast_to(b2.reshape(1, D), (8, D))
    b3b = jnp.broadcast_to(b3.reshape(1, D), (8, D))

    hist_s, hist_d = _make_deg(N, D, EPW)(srcd, dstd, zN)
    cnt = pl.pallas_call(
        _dred_body,
        grid=(1,),
        in_specs=[pl.BlockSpec((NW, N), lambda i: (0, 0)),
                  pl.BlockSpec((NW, N), lambda i: (0, 0)),
                  pl.BlockSpec((NW, D), lambda i: (0, 0)),
                  pl.BlockSpec((NW, D), lambda i: (0, 0))],
        out_specs=pl.BlockSpec((N, D), lambda i: (0, 0)),
        out_shape=jax.ShapeDtypeStruct((N, D), jnp.float32),
    )(hist_s.reshape(NW, N), hist_d.reshape(NW, N), sel_s, sel_d)
    agg_fn = _make_agg(N, D, CHA, KA)

    spec_c = pl.BlockSpec((BR, D), lambda i: (i, 0))
    spec_a = pl.BlockSpec((NC, BR, D), lambda i: (0, i, 0))
    spec_w = pl.BlockSpec((D, D), lambda i: (0, 0))
    spec_b = pl.BlockSpec((8, D), lambda i: (0, 0))
    spec_f = pl.BlockSpec((BR, D), lambda i: (i, 0))

    hw = _tc_call(_stage0_body, N, D, BR, [spec_f, spec_c, spec_w])(
        features, cnt, W1)
    for W, bb in ((W2, b1b), (W3, b2b)):
        a = agg_fn(hw, srca, dsta, zND)
        hw = _tc_call(_stage_mid_body, N, D, BR,
                      [spec_a, spec_c, spec_b, spec_w])(a, cnt, bb, W)
    a = agg_fn(hw, srca, dsta, zND)
    out = _tc_call(_stage_last_body, N, D, BR,
                   [spec_a, spec_c, spec_b])(a, cnt, b3b)
    return out


# speedup vs baseline: 7.8601x; 1.0317x over previous
import dataclasses
import functools

import jax
import jax.numpy as jnp
from jax import lax
from jax.experimental import pallas as pl
from jax.experimental.pallas import tpu as pltpu
from jax.experimental.pallas import tpu_sc as plsc

NC = 2
NS = 16
NW = NC * NS
KA = 125


def _sc_mesh():
    return plsc.VectorSubcoreMesh(core_axis_name="c", subcore_axis_name="s",
                                  num_cores=NC, num_subcores=NS)


def _stripe(sid, N):
    rps8 = (N // NS) // 8 * 8
    tail = N - NS * rps8
    return sid * rps8, rps8, NS * rps8, tail


def _make_deg(N, D, EPW):
    R, G = EPW // 80, 80 // 16

    @functools.partial(
        pl.kernel,
        out_type=(jax.ShapeDtypeStruct((NW, 1, N), jnp.float32),
                  jax.ShapeDtypeStruct((NW, 1, N), jnp.float32)),
        mesh=_sc_mesh(),
        scratch_types=[
            pltpu.VMEM((N,), jnp.float32),
            pltpu.VMEM((N,), jnp.float32),
            pltpu.VMEM((R, 80), jnp.int32),
            pltpu.VMEM((R, 80), jnp.int32),
        ],
        compiler_params=dataclasses.replace(pltpu.CompilerParams(),
                                            needs_layout_passes=False),
    )
    def deg(src_hbm, dst_hbm, z_hbm, outs_hbm, outd_hbm,
            hs, hd, isv, idv):
        cid = lax.axis_index("c")
        sid = lax.axis_index("s")
        wid = cid * NS + sid
        pltpu.sync_copy(z_hbm, hs)
        pltpu.sync_copy(z_hbm, hd)
        pltpu.sync_copy(src_hbm.at[wid], isv)
        pltpu.sync_copy(dst_hbm.at[wid], idv)
        ones16 = jnp.full((16,), 1.0, jnp.float32)

        @pl.loop(0, R)
        def _(r):
            for j in range(G):
                vs = isv[r, pl.ds(j * 16, 16)]
                plsc.addupdate_scatter(hs, [vs], ones16)
                vd = idv[r, pl.ds(j * 16, 16)]
                plsc.addupdate_scatter(hd, [vd], ones16)

        pltpu.sync_copy(hs, outs_hbm.at[wid, 0])
        pltpu.sync_copy(hd, outd_hbm.at[wid, 0])

    return deg


def _dred_body(hs_ref, hd_ref, ss_ref, sd_ref, o_ref):
    dn = (((0,), (0,)), ((), ()))
    acc = lax.dot_general(hs_ref[...], ss_ref[...], dn,
                          precision=lax.Precision.HIGHEST)
    acc = acc + lax.dot_general(hd_ref[...], sd_ref[...], dn,
                                precision=lax.Precision.HIGHEST)
    o_ref[...] = lax.rsqrt(jnp.maximum(acc, 1.0))


def _make_agg(N, D, CH, K):
    @functools.partial(
        pl.kernel,
        out_type=jax.ShapeDtypeStruct((NC, N, D), jnp.float32),
        mesh=_sc_mesh(),
        scratch_types=[
            pltpu.VMEM_SHARED((N, D), jnp.float32),
            pltpu.VMEM((CH, K), jnp.int32),
            pltpu.VMEM((CH, K), jnp.int32),
            pltpu.VMEM((K, D), jnp.float32),
            pltpu.SemaphoreType.DMA,
        ],
    )
    def agg(hw_hbm, src_hbm, dst_hbm, z_hbm, out_hbm,
            acc_sh, isv, idv, rows, psem):
        cid = lax.axis_index("c")
        sid = lax.axis_index("s")
        wid = cid * NS + sid
        r0, rn, t0, tn = _stripe(sid, N)
        pltpu.async_copy(z_hbm.at[pl.ds(r0, rn)], acc_sh.at[pl.ds(r0, rn)],
                         psem)
        pltpu.async_copy(src_hbm.at[wid], isv, psem)
        pltpu.async_copy(dst_hbm.at[wid], idv, psem)

        @pl.when(jnp.logical_and(sid == 0, tn > 0))
        def _():
            pltpu.async_copy(z_hbm.at[pl.ds(t0, tn)], acc_sh.at[pl.ds(t0, tn)],
                             psem).wait()

        pltpu.make_async_copy(z_hbm.at[pl.ds(r0, rn)],
                              acc_sh.at[pl.ds(r0, rn)], psem).wait()
        pltpu.make_async_copy(src_hbm.at[wid], isv, psem).wait()
        pltpu.make_async_copy(dst_hbm.at[wid], idv, psem).wait()
        plsc.subcore_barrier()

        @pl.loop(0, CH)
        def _(t):
            pltpu.sync_copy(hw_hbm.at[isv.at[t]], rows)
            pltpu.sync_copy(rows, acc_sh.at[idv.at[t]], add=True)

        plsc.subcore_barrier()
        pltpu.sync_copy(acc_sh.at[pl.ds(r0, rn)],
                        out_hbm.at[cid, pl.ds(r0, rn)])

        @pl.when(jnp.logical_and(sid == 0, tn > 0))
        def _():
            pltpu.sync_copy(acc_sh.at[pl.ds(t0, tn)],
                            out_hbm.at[cid, pl.ds(t0, tn)])

    return agg


def _dcol(c_ref, col):
    return c_ref[:, col:col + 1]


def _dout(c_ref):
    return _dcol(c_ref, 0)


def _din(c_ref):
    return _dcol(c_ref, c_ref.shape[-1] // 2)


def _stage0_body(f_ref, c_ref, w_ref, o_ref):
    h = f_ref[...] * _dout(c_ref)
    o_ref[...] = lax.dot(h, w_ref[...], precision=lax.Precision.HIGHEST)


def _stage_mid_body(a_ref, c_ref, b_ref, w_ref, o_ref):
    a = a_ref[0] + a_ref[1]
    h = jnp.maximum(a * _din(c_ref) + b_ref[0:1, :], 0.0)
    o_ref[...] = lax.dot(h * _dout(c_ref), w_ref[...],
                         precision=lax.Precision.HIGHEST)


def _stage_last_body(a_ref, c_ref, b_ref, o_ref):
    a = a_ref[0] + a_ref[1]
    o_ref[...] = jnp.maximum(a * _din(c_ref) + b_ref[0:1, :], 0.0)


def _tc_call(body, N, D, BR, in_specs):
    return pl.pallas_call(
        body,
        grid=(N // BR,),
        in_specs=in_specs,
        out_specs=pl.BlockSpec((BR, D), lambda i: (i, 0)),
        out_shape=jax.ShapeDtypeStruct((N, D), jnp.float32),
    )


def kernel(features, edge_index, W1, b1, W2, b2, W3, b3):
    N, D = features.shape
    E = edge_index.shape[1]
    CHA = E // (NW * KA)
    BR = 2000

    EPW = E // NW
    srcd = edge_index[0].reshape(NW, EPW // 80, 80)
    dstd = edge_index[1].reshape(NW, EPW // 80, 80)
    srca = edge_index[0].reshape(NW, CHA, KA)
    dsta = edge_index[1].reshape(NW, CHA, KA)
    zND = jnp.zeros((N, D), jnp.float32)
    zN = jnp.zeros((N,), jnp.float32)
    lane = jnp.arange(D) < (D // 2)
    sel_s = jnp.broadcast_to(jnp.where(lane, 1.0, 0.0), (NW, D))
    sel_s = sel_s.astype(jnp.float32)
    sel_d = jnp.broadcast_to(jnp.where(lane, 0.0, 1.0), (NW, D))
    sel_d = sel_d.astype(jnp.float32)
    b1b = jnp.broadcast_to(b1.reshape(1, D), (8, D))
    b2b = jnp.broadcast_to(b2.reshape(1, D), (8, D))
    b3b = jnp.broadcast_to(b3.reshape(1, D), (8, D))

    hist_s, hist_d = _make_deg(N, D, EPW)(srcd, dstd, zN)
    cnt = pl.pallas_call(
        _dred_body,
        grid=(1,),
        in_specs=[pl.BlockSpec((NW, N), lambda i: (0, 0)),
                  pl.BlockSpec((NW, N), lambda i: (0, 0)),
                  pl.BlockSpec((NW, D), lambda i: (0, 0)),
                  pl.BlockSpec((NW, D), lambda i: (0, 0))],
        out_specs=pl.BlockSpec((N, D), lambda i: (0, 0)),
        out_shape=jax.ShapeDtypeStruct((N, D), jnp.float32),
    )(hist_s.reshape(NW, N), hist_d.reshape(NW, N), sel_s, sel_d)
    agg_fn = _make_agg(N, D, CHA, KA)

    spec_c = pl.BlockSpec((BR, D), lambda i: (i, 0))
    spec_a = pl.BlockSpec((NC, BR, D), lambda i: (0, i, 0))
    spec_w = pl.BlockSpec((D, D), lambda i: (0, 0))
    spec_b = pl.BlockSpec((8, D), lambda i: (0, 0))
    spec_f = pl.BlockSpec((BR, D), lambda i: (i, 0))

    hw = _tc_call(_stage0_body, N, D, BR, [spec_f, spec_c, spec_w])(
        features, cnt, W1)
    for W, bb in ((W2, b1b), (W3, b2b)):
        a = agg_fn(hw, srca, dsta, zND)
        hw = _tc_call(_stage_mid_body, N, D, BR,
                      [spec_a, spec_c, spec_b, spec_w])(a, cnt, bb, W)
    a = agg_fn(hw, srca, dsta, zND)
    out = _tc_call(_stage_last_body, N, D, BR,
                   [spec_a, spec_c, spec_b])(a, cnt, b3b)
    return out


# speedup vs baseline: 8.7021x; 1.1071x over previous
import dataclasses
import functools

import jax
import jax.numpy as jnp
from jax import lax
from jax.experimental import pallas as pl
from jax.experimental.pallas import tpu as pltpu
from jax.experimental.pallas import tpu_sc as plsc

NC = 2
NS = 16
NW = NC * NS
KA = 200


def _sc_mesh():
    return plsc.VectorSubcoreMesh(core_axis_name="c", subcore_axis_name="s",
                                  num_cores=NC, num_subcores=NS)


def _stripe(sid, N):
    rps8 = (N // NS) // 8 * 8
    tail = N - NS * rps8
    return sid * rps8, rps8, NS * rps8, tail


def _make_deg(N, D, EPW):
    R, G = EPW // 80, 80 // 16

    @functools.partial(
        pl.kernel,
        out_type=(jax.ShapeDtypeStruct((NW, 1, N), jnp.float32),
                  jax.ShapeDtypeStruct((NW, 1, N), jnp.float32)),
        mesh=_sc_mesh(),
        scratch_types=[
            pltpu.VMEM((N,), jnp.float32),
            pltpu.VMEM((N,), jnp.float32),
            pltpu.VMEM((R, 80), jnp.int32),
            pltpu.VMEM((R, 80), jnp.int32),
        ],
        compiler_params=dataclasses.replace(pltpu.CompilerParams(),
                                            needs_layout_passes=False),
    )
    def deg(src_hbm, dst_hbm, z_hbm, outs_hbm, outd_hbm,
            hs, hd, isv, idv):
        cid = lax.axis_index("c")
        sid = lax.axis_index("s")
        wid = cid * NS + sid
        pltpu.sync_copy(z_hbm, hs)
        pltpu.sync_copy(z_hbm, hd)
        pltpu.sync_copy(src_hbm.at[wid], isv)
        pltpu.sync_copy(dst_hbm.at[wid], idv)
        ones16 = jnp.full((16,), 1.0, jnp.float32)

        @pl.loop(0, R)
        def _(r):
            for j in range(G):
                vs = isv[r, pl.ds(j * 16, 16)]
                plsc.addupdate_scatter(hs, [vs], ones16)
                vd = idv[r, pl.ds(j * 16, 16)]
                plsc.addupdate_scatter(hd, [vd], ones16)

        pltpu.sync_copy(hs, outs_hbm.at[wid, 0])
        pltpu.sync_copy(hd, outd_hbm.at[wid, 0])

    return deg


def _dred_body(hs_ref, hd_ref, ss_ref, sd_ref, o_ref):
    dn = (((0,), (0,)), ((), ()))
    acc = lax.dot_general(hs_ref[...], ss_ref[...], dn,
                          precision=lax.Precision.HIGHEST)
    acc = acc + lax.dot_general(hd_ref[...], sd_ref[...], dn,
                                precision=lax.Precision.HIGHEST)
    o_ref[...] = lax.rsqrt(jnp.maximum(acc, 1.0))


def _make_agg(N, D, CH, K, EPW):
    @functools.partial(
        pl.kernel,
        out_type=jax.ShapeDtypeStruct((NC, N, D), jnp.float32),
        mesh=_sc_mesh(),
        scratch_types=[
            pltpu.VMEM_SHARED((N, D), jnp.float32),
            pltpu.VMEM((EPW,), jnp.int32),
            pltpu.VMEM((EPW,), jnp.int32),
            pltpu.VMEM((K, D), jnp.float32),
            pltpu.SemaphoreType.DMA,
        ],
    )
    def agg(hw_hbm, src_hbm, dst_hbm, z_hbm, out_hbm,
            acc_sh, isv, idv, rows, psem):
        cid = lax.axis_index("c")
        sid = lax.axis_index("s")
        wid = cid * NS + sid
        e0 = wid * EPW
        r0, rn, t0, tn = _stripe(sid, N)
        pltpu.async_copy(z_hbm.at[pl.ds(r0, rn)], acc_sh.at[pl.ds(r0, rn)],
                         psem)
        pltpu.async_copy(src_hbm.at[pl.ds(e0, EPW)], isv, psem)
        pltpu.async_copy(dst_hbm.at[pl.ds(e0, EPW)], idv, psem)

        @pl.when(jnp.logical_and(sid == 0, tn > 0))
        def _():
            pltpu.async_copy(z_hbm.at[pl.ds(t0, tn)], acc_sh.at[pl.ds(t0, tn)],
                             psem).wait()

        pltpu.make_async_copy(z_hbm.at[pl.ds(r0, rn)],
                              acc_sh.at[pl.ds(r0, rn)], psem).wait()
        pltpu.make_async_copy(src_hbm.at[pl.ds(e0, EPW)], isv, psem).wait()
        pltpu.make_async_copy(dst_hbm.at[pl.ds(e0, EPW)], idv, psem).wait()
        plsc.subcore_barrier()

        @pl.loop(0, CH)
        def _(t):
            pltpu.sync_copy(hw_hbm.at[isv.at[pl.ds(t * K, K)]], rows)
            pltpu.sync_copy(rows, acc_sh.at[idv.at[pl.ds(t * K, K)]],
                            add=True)

        plsc.subcore_barrier()
        pltpu.sync_copy(acc_sh.at[pl.ds(r0, rn)],
                        out_hbm.at[cid, pl.ds(r0, rn)])

        @pl.when(jnp.logical_and(sid == 0, tn > 0))
        def _():
            pltpu.sync_copy(acc_sh.at[pl.ds(t0, tn)],
                            out_hbm.at[cid, pl.ds(t0, tn)])

    return agg


def _dcol(c_ref, col):
    return c_ref[:, col:col + 1]


def _dout(c_ref):
    return _dcol(c_ref, 0)


def _din(c_ref):
    return _dcol(c_ref, c_ref.shape[-1] // 2)


def _stage0_body(f_ref, c_ref, w_ref, o_ref):
    h = f_ref[...] * _dout(c_ref)
    o_ref[...] = lax.dot(h, w_ref[...], precision=lax.Precision.HIGHEST)


def _stage_mid_body(a_ref, c_ref, b_ref, w_ref, o_ref):
    a = a_ref[0] + a_ref[1]
    h = jnp.maximum(a * _din(c_ref) + b_ref[0:1, :], 0.0)
    o_ref[...] = lax.dot(h * _dout(c_ref), w_ref[...],
                         precision=lax.Precision.HIGHEST)


def _stage_last_body(a_ref, c_ref, b_ref, o_ref):
    a = a_ref[0] + a_ref[1]
    o_ref[...] = jnp.maximum(a * _din(c_ref) + b_ref[0:1, :], 0.0)


def _tc_call(body, N, D, BR, in_specs):
    return pl.pallas_call(
        body,
        grid=(N // BR,),
        in_specs=in_specs,
        out_specs=pl.BlockSpec((BR, D), lambda i: (i, 0)),
        out_shape=jax.ShapeDtypeStruct((N, D), jnp.float32),
    )


def kernel(features, edge_index, W1, b1, W2, b2, W3, b3):
    N, D = features.shape
    E = edge_index.shape[1]
    CHA = E // (NW * KA)
    EPWA = E // NW
    BR = 2000

    EPW = E // NW
    srcd = edge_index[0].reshape(NW, EPW // 80, 80)
    dstd = edge_index[1].reshape(NW, EPW // 80, 80)
    srca = edge_index[0]
    dsta = edge_index[1]
    zND = jnp.zeros((N, D), jnp.float32)
    zN = jnp.zeros((N,), jnp.float32)
    lane = jnp.arange(D) < (D // 2)
    sel_s = jnp.broadcast_to(jnp.where(lane, 1.0, 0.0), (NW, D))
    sel_s = sel_s.astype(jnp.float32)
    sel_d = jnp.broadcast_to(jnp.where(lane, 0.0, 1.0), (NW, D))
    sel_d = sel_d.astype(jnp.float32)
    b1b = jnp.broadcast_to(b1.reshape(1, D), (8, D))
    b2b = jnp.broadcast_to(b2.reshape(1, D), (8, D))
    b3b = jnp.broadcast_to(b3.reshape(1, D), (8, D))

    hist_s, hist_d = _make_deg(N, D, EPW)(srcd, dstd, zN)
    cnt = pl.pallas_call(
        _dred_body,
        grid=(1,),
        in_specs=[pl.BlockSpec((NW, N), lambda i: (0, 0)),
                  pl.BlockSpec((NW, N), lambda i: (0, 0)),
                  pl.BlockSpec((NW, D), lambda i: (0, 0)),
                  pl.BlockSpec((NW, D), lambda i: (0, 0))],
        out_specs=pl.BlockSpec((N, D), lambda i: (0, 0)),
        out_shape=jax.ShapeDtypeStruct((N, D), jnp.float32),
    )(hist_s.reshape(NW, N), hist_d.reshape(NW, N), sel_s, sel_d)
    agg_fn = _make_agg(N, D, CHA // 1, KA, EPWA)

    spec_c = pl.BlockSpec((BR, D), lambda i: (i, 0))
    spec_a = pl.BlockSpec((NC, BR, D), lambda i: (0, i, 0))
    spec_w = pl.BlockSpec((D, D), lambda i: (0, 0))
    spec_b = pl.BlockSpec((8, D), lambda i: (0, 0))
    spec_f = pl.BlockSpec((BR, D), lambda i: (i, 0))

    hw = _tc_call(_stage0_body, N, D, BR, [spec_f, spec_c, spec_w])(
        features, cnt, W1)
    for W, bb in ((W2, b1b), (W3, b2b)):
        a = agg_fn(hw, srca, dsta, zND)
        hw = _tc_call(_stage_mid_body, N, D, BR,
                      [spec_a, spec_c, spec_b, spec_w])(a, cnt, bb, W)
    a = agg_fn(hw, srca, dsta, zND)
    out = _tc_call(_stage_last_body, N, D, BR,
                   [spec_a, spec_c, spec_b])(a, cnt, b3b)
    return out


# speedup vs baseline: 10.7256x; 1.2325x over previous
import dataclasses
import functools

import jax
import jax.numpy as jnp
from jax import lax
from jax.experimental import pallas as pl
from jax.experimental.pallas import tpu as pltpu
from jax.experimental.pallas import tpu_sc as plsc

NC = 2
NS = 16
NW = NC * NS
KA = 80


def _sc_mesh():
    return plsc.VectorSubcoreMesh(core_axis_name="c", subcore_axis_name="s",
                                  num_cores=NC, num_subcores=NS)


def _stripe(sid, N):
    rps8 = (N // NS) // 8 * 8
    tail = N - NS * rps8
    return sid * rps8, rps8, NS * rps8, tail


def _make_deg(N, D, EPW):
    R, G = EPW // 80, 80 // 16

    @functools.partial(
        pl.kernel,
        out_type=(jax.ShapeDtypeStruct((NW, 1, N), jnp.float32),
                  jax.ShapeDtypeStruct((NW, 1, N), jnp.float32)),
        mesh=_sc_mesh(),
        scratch_types=[
            pltpu.VMEM((N,), jnp.float32),
            pltpu.VMEM((N,), jnp.float32),
            pltpu.VMEM((R, 80), jnp.int32),
            pltpu.VMEM((R, 80), jnp.int32),
        ],
        compiler_params=dataclasses.replace(pltpu.CompilerParams(),
                                            needs_layout_passes=False),
    )
    def deg(src_hbm, dst_hbm, z_hbm, outs_hbm, outd_hbm,
            hs, hd, isv, idv):
        cid = lax.axis_index("c")
        sid = lax.axis_index("s")
        wid = cid * NS + sid
        pltpu.sync_copy(z_hbm, hs)
        pltpu.sync_copy(z_hbm, hd)
        pltpu.sync_copy(src_hbm.at[wid], isv)
        pltpu.sync_copy(dst_hbm.at[wid], idv)
        ones16 = jnp.full((16,), 1.0, jnp.float32)

        @pl.loop(0, R)
        def _(r):
            for j in range(G):
                vs = isv[r, pl.ds(j * 16, 16)]
                plsc.addupdate_scatter(hs, [vs], ones16)
                vd = idv[r, pl.ds(j * 16, 16)]
                plsc.addupdate_scatter(hd, [vd], ones16)

        pltpu.sync_copy(hs, outs_hbm.at[wid, 0])
        pltpu.sync_copy(hd, outd_hbm.at[wid, 0])

    return deg


def _dred_body(hs_ref, hd_ref, ss_ref, sd_ref, o_ref):
    dn = (((0,), (0,)), ((), ()))
    acc = lax.dot_general(hs_ref[...], ss_ref[...], dn,
                          precision=lax.Precision.HIGHEST)
    acc = acc + lax.dot_general(hd_ref[...], sd_ref[...], dn,
                                precision=lax.Precision.HIGHEST)
    o_ref[...] = lax.rsqrt(jnp.maximum(acc, 1.0))


def _make_agg(N, D, CH, K, EPW):
    @functools.partial(
        pl.kernel,
        out_type=jax.ShapeDtypeStruct((NC, N, D), jnp.float32),
        mesh=_sc_mesh(),
        scratch_types=[
            pltpu.VMEM_SHARED((N, D), jnp.float32),
            pltpu.VMEM((EPW,), jnp.int32),
            pltpu.VMEM((EPW,), jnp.int32),
            pltpu.VMEM((K, D), jnp.float32),
            pltpu.VMEM((K, D), jnp.float32),
            pltpu.SemaphoreType.DMA,
            pltpu.SemaphoreType.DMA,
            pltpu.SemaphoreType.DMA,
        ],
    )
    def agg(hw_hbm, src_hbm, dst_hbm, z_hbm, out_hbm,
            acc_sh, isv, idv, rows_a, rows_b, gsa, gsb, psem):
        cid = lax.axis_index("c")
        sid = lax.axis_index("s")
        wid = cid * NS + sid
        e0 = wid * EPW
        r0, rn, t0, tn = _stripe(sid, N)
        pltpu.async_copy(z_hbm.at[pl.ds(r0, rn)], acc_sh.at[pl.ds(r0, rn)],
                         psem)
        pltpu.async_copy(src_hbm.at[pl.ds(e0, EPW)], isv, psem)
        pltpu.async_copy(dst_hbm.at[pl.ds(e0, EPW)], idv, psem)

        @pl.when(jnp.logical_and(sid == 0, tn > 0))
        def _():
            pltpu.async_copy(z_hbm.at[pl.ds(t0, tn)], acc_sh.at[pl.ds(t0, tn)],
                             psem).wait()

        pltpu.make_async_copy(z_hbm.at[pl.ds(r0, rn)],
                              acc_sh.at[pl.ds(r0, rn)], psem).wait()
        pltpu.make_async_copy(src_hbm.at[pl.ds(e0, EPW)], isv, psem).wait()
        pltpu.make_async_copy(dst_hbm.at[pl.ds(e0, EPW)], idv, psem).wait()
        plsc.subcore_barrier()

        @pl.loop(0, CH + 2, step=2)
        def _(t):
            @pl.when(t >= 2)
            def _():
                pltpu.make_async_copy(hw_hbm.at[isv.at[pl.ds((t - 2) * K, K)]],
                                      rows_a, gsa).wait()
                pltpu.sync_copy(rows_a, acc_sh.at[idv.at[pl.ds((t - 2) * K, K)]],
                                add=True)

            @pl.when(t < CH)
            def _():
                pltpu.async_copy(hw_hbm.at[isv.at[pl.ds(t * K, K)]],
                                 rows_a, gsa)

            @pl.when(jnp.logical_and(t >= 1, t - 1 < CH))
            def _():
                pltpu.make_async_copy(hw_hbm.at[isv.at[pl.ds((t - 1) * K, K)]],
                                      rows_b, gsb).wait()
                pltpu.sync_copy(rows_b, acc_sh.at[idv.at[pl.ds((t - 1) * K, K)]],
                                add=True)

            @pl.when(t + 1 < CH)
            def _():
                pltpu.async_copy(hw_hbm.at[isv.at[pl.ds((t + 1) * K, K)]],
                                 rows_b, gsb)

        plsc.subcore_barrier()
        pltpu.sync_copy(acc_sh.at[pl.ds(r0, rn)],
                        out_hbm.at[cid, pl.ds(r0, rn)])

        @pl.when(jnp.logical_and(sid == 0, tn > 0))
        def _():
            pltpu.sync_copy(acc_sh.at[pl.ds(t0, tn)],
                            out_hbm.at[cid, pl.ds(t0, tn)])

    return agg


def _dcol(c_ref, col):
    return c_ref[:, col:col + 1]


def _dout(c_ref):
    return _dcol(c_ref, 0)


def _din(c_ref):
    return _dcol(c_ref, c_ref.shape[-1] // 2)


def _stage0_body(f_ref, c_ref, w_ref, o_ref):
    h = f_ref[...] * _dout(c_ref)
    o_ref[...] = lax.dot(h, w_ref[...], precision=lax.Precision.HIGHEST)


def _stage_mid_body(a_ref, c_ref, b_ref, w_ref, o_ref):
    a = a_ref[0] + a_ref[1]
    h = jnp.maximum(a * _din(c_ref) + b_ref[0:1, :], 0.0)
    o_ref[...] = lax.dot(h * _dout(c_ref), w_ref[...],
                         precision=lax.Precision.HIGHEST)


def _stage_last_body(a_ref, c_ref, b_ref, o_ref):
    a = a_ref[0] + a_ref[1]
    o_ref[...] = jnp.maximum(a * _din(c_ref) + b_ref[0:1, :], 0.0)


def _tc_call(body, N, D, BR, in_specs):
    return pl.pallas_call(
        body,
        grid=(N // BR,),
        in_specs=in_specs,
        out_specs=pl.BlockSpec((BR, D), lambda i: (i, 0)),
        out_shape=jax.ShapeDtypeStruct((N, D), jnp.float32),
    )


def kernel(features, edge_index, W1, b1, W2, b2, W3, b3):
    N, D = features.shape
    E = edge_index.shape[1]
    CHA = E // (NW * KA)
    EPWA = E // NW
    BR = 2000

    EPW = E // NW
    srcd = edge_index[0].reshape(NW, EPW // 80, 80)
    dstd = edge_index[1].reshape(NW, EPW // 80, 80)
    srca = edge_index[0]
    dsta = edge_index[1]
    zND = jnp.zeros((N, D), jnp.float32)
    zN = jnp.zeros((N,), jnp.float32)
    lane = jnp.arange(D) < (D // 2)
    sel_s = jnp.broadcast_to(jnp.where(lane, 1.0, 0.0), (NW, D))
    sel_s = sel_s.astype(jnp.float32)
    sel_d = jnp.broadcast_to(jnp.where(lane, 0.0, 1.0), (NW, D))
    sel_d = sel_d.astype(jnp.float32)
    b1b = jnp.broadcast_to(b1.reshape(1, D), (8, D))
    b2b = jnp.broadcast_to(b2.reshape(1, D), (8, D))
    b3b = jnp.broadcast_to(b3.reshape(1, D), (8, D))

    hist_s, hist_d = _make_deg(N, D, EPW)(srcd, dstd, zN)
    cnt = pl.pallas_call(
        _dred_body,
        grid=(1,),
        in_specs=[pl.BlockSpec((NW, N), lambda i: (0, 0)),
                  pl.BlockSpec((NW, N), lambda i: (0, 0)),
                  pl.BlockSpec((NW, D), lambda i: (0, 0)),
                  pl.BlockSpec((NW, D), lambda i: (0, 0))],
        out_specs=pl.BlockSpec((N, D), lambda i: (0, 0)),
        out_shape=jax.ShapeDtypeStruct((N, D), jnp.float32),
    )(hist_s.reshape(NW, N), hist_d.reshape(NW, N), sel_s, sel_d)
    agg_fn = _make_agg(N, D, CHA // 1, KA, EPWA)

    spec_c = pl.BlockSpec((BR, D), lambda i: (i, 0))
    spec_a = pl.BlockSpec((NC, BR, D), lambda i: (0, i, 0))
    spec_w = pl.BlockSpec((D, D), lambda i: (0, 0))
    spec_b = pl.BlockSpec((8, D), lambda i: (0, 0))
    spec_f = pl.BlockSpec((BR, D), lambda i: (i, 0))

    hw = _tc_call(_stage0_body, N, D, BR, [spec_f, spec_c, spec_w])(
        features, cnt, W1)
    for W, bb in ((W2, b1b), (W3, b2b)):
        a = agg_fn(hw, srca, dsta, zND)
        hw = _tc_call(_stage_mid_body, N, D, BR,
                      [spec_a, spec_c, spec_b, spec_w])(a, cnt, bb, W)
    a = agg_fn(hw, srca, dsta, zND)
    out = _tc_call(_stage_last_body, N, D, BR,
                   [spec_a, spec_c, spec_b])(a, cnt, b3b)
    return out


# speedup vs baseline: 10.8624x; 1.0128x over previous
import dataclasses
import functools

import jax
import jax.numpy as jnp
from jax import lax
from jax.experimental import pallas as pl
from jax.experimental.pallas import tpu as pltpu
from jax.experimental.pallas import tpu_sc as plsc

NC = 2
NS = 16
NW = NC * NS
KA = 80


def _sc_mesh():
    return plsc.VectorSubcoreMesh(core_axis_name="c", subcore_axis_name="s",
                                  num_cores=NC, num_subcores=NS)


def _stripe(sid, N):
    rps8 = (N // NS) // 8 * 8
    tail = N - NS * rps8
    return sid * rps8, rps8, NS * rps8, tail


def _make_deg(N, D, EPW):
    R, G = EPW // 80, 80 // 16

    @functools.partial(
        pl.kernel,
        out_type=(jax.ShapeDtypeStruct((NW, 1, N), jnp.float32),
                  jax.ShapeDtypeStruct((NW, 1, N), jnp.float32)),
        mesh=_sc_mesh(),
        scratch_types=[
            pltpu.VMEM((N,), jnp.float32),
            pltpu.VMEM((N,), jnp.float32),
            pltpu.VMEM((R, 80), jnp.int32),
            pltpu.VMEM((R, 80), jnp.int32),
        ],
        compiler_params=dataclasses.replace(pltpu.CompilerParams(),
                                            needs_layout_passes=False),
    )
    def deg(src_hbm, dst_hbm, z_hbm, outs_hbm, outd_hbm,
            hs, hd, isv, idv):
        cid = lax.axis_index("c")
        sid = lax.axis_index("s")
        wid = cid * NS + sid
        pltpu.sync_copy(z_hbm, hs)
        pltpu.sync_copy(z_hbm, hd)
        pltpu.sync_copy(src_hbm.at[wid], isv)
        pltpu.sync_copy(dst_hbm.at[wid], idv)
        ones16 = jnp.full((16,), 1.0, jnp.float32)

        @pl.loop(0, R)
        def _(r):
            for j in range(G):
                vs = isv[r, pl.ds(j * 16, 16)]
                plsc.addupdate_scatter(hs, [vs], ones16)
                vd = idv[r, pl.ds(j * 16, 16)]
                plsc.addupdate_scatter(hd, [vd], ones16)

        pltpu.sync_copy(hs, outs_hbm.at[wid, 0])
        pltpu.sync_copy(hd, outd_hbm.at[wid, 0])

    return deg


def _dred_body(hs_ref, hd_ref, ss_ref, sd_ref, o_ref):
    dn = (((0,), (0,)), ((), ()))
    acc = lax.dot_general(hs_ref[...], ss_ref[...], dn,
                          precision=lax.Precision.HIGHEST)
    acc = acc + lax.dot_general(hd_ref[...], sd_ref[...], dn,
                                precision=lax.Precision.HIGHEST)
    o_ref[...] = lax.rsqrt(jnp.maximum(acc, 1.0))


def _make_agg(N, D, CH, K, EPW):
    @functools.partial(
        pl.kernel,
        out_type=jax.ShapeDtypeStruct((NC, N, D), jnp.float32),
        mesh=_sc_mesh(),
        scratch_types=[
            pltpu.VMEM_SHARED((N, D), jnp.float32),
            pltpu.VMEM((EPW,), jnp.int32),
            pltpu.VMEM((EPW,), jnp.int32),
            pltpu.VMEM((K, D), jnp.float32),
            pltpu.VMEM((K, D), jnp.float32),
            pltpu.SemaphoreType.DMA,
            pltpu.SemaphoreType.DMA,
            pltpu.SemaphoreType.DMA,
        ],
    )
    def agg(hw_hbm, src_hbm, dst_hbm, z_hbm, out_hbm,
            acc_sh, isv, idv, rows_a, rows_b, gsa, gsb, psem):
        cid = lax.axis_index("c")
        sid = lax.axis_index("s")
        wid = cid * NS + sid
        e0 = wid * EPW
        r0, rn, t0, tn = _stripe(sid, N)
        pltpu.async_copy(z_hbm.at[pl.ds(r0, rn)], acc_sh.at[pl.ds(r0, rn)],
                         psem)
        pltpu.async_copy(src_hbm.at[pl.ds(e0, EPW)], isv, psem)
        pltpu.async_copy(dst_hbm.at[pl.ds(e0, EPW)], idv, psem)

        @pl.when(jnp.logical_and(sid == 0, tn > 0))
        def _():
            pltpu.async_copy(z_hbm.at[pl.ds(t0, tn)], acc_sh.at[pl.ds(t0, tn)],
                             psem).wait()

        pltpu.make_async_copy(z_hbm.at[pl.ds(r0, rn)],
                              acc_sh.at[pl.ds(r0, rn)], psem).wait()
        pltpu.make_async_copy(src_hbm.at[pl.ds(e0, EPW)], isv, psem).wait()
        pltpu.make_async_copy(dst_hbm.at[pl.ds(e0, EPW)], idv, psem).wait()
        plsc.subcore_barrier()

        @pl.loop(0, CH + 2, step=2)
        def _(t):
            @pl.when(t >= 2)
            def _():
                pltpu.make_async_copy(hw_hbm.at[isv.at[pl.ds((t - 2) * K, K)]],
                                      rows_a, gsa).wait()
                pltpu.sync_copy(rows_a, acc_sh.at[idv.at[pl.ds((t - 2) * K, K)]],
                                add=True)

            @pl.when(t < CH)
            def _():
                pltpu.async_copy(hw_hbm.at[isv.at[pl.ds(t * K, K)]],
                                 rows_a, gsa)

            @pl.when(jnp.logical_and(t >= 1, t - 1 < CH))
            def _():
                pltpu.make_async_copy(hw_hbm.at[isv.at[pl.ds((t - 1) * K, K)]],
                                      rows_b, gsb).wait()
                pltpu.sync_copy(rows_b, acc_sh.at[idv.at[pl.ds((t - 1) * K, K)]],
                                add=True)

            @pl.when(t + 1 < CH)
            def _():
                pltpu.async_copy(hw_hbm.at[isv.at[pl.ds((t + 1) * K, K)]],
                                 rows_b, gsb)

        plsc.subcore_barrier()
        pltpu.sync_copy(acc_sh.at[pl.ds(r0, rn)],
                        out_hbm.at[cid, pl.ds(r0, rn)])

        @pl.when(jnp.logical_and(sid == 0, tn > 0))
        def _():
            pltpu.sync_copy(acc_sh.at[pl.ds(t0, tn)],
                            out_hbm.at[cid, pl.ds(t0, tn)])

    return agg


def _dcol(c_ref, col):
    return c_ref[:, col:col + 1]


def _dout(c_ref):
    return _dcol(c_ref, 0)


def _din(c_ref):
    return _dcol(c_ref, c_ref.shape[-1] // 2)


def _stage0_body(f_ref, c_ref, w_ref, o_ref):
    h = f_ref[...] * _dout(c_ref)
    o_ref[...] = lax.dot(h, w_ref[...])


def _stage_mid_body(a_ref, c_ref, b_ref, w_ref, o_ref):
    a = a_ref[0] + a_ref[1]
    h = jnp.maximum(a * _din(c_ref) + b_ref[0:1, :], 0.0)
    o_ref[...] = lax.dot(h * _dout(c_ref), w_ref[...])


def _stage_last_body(a_ref, c_ref, b_ref, o_ref):
    a = a_ref[0] + a_ref[1]
    o_ref[...] = jnp.maximum(a * _din(c_ref) + b_ref[0:1, :], 0.0)


def _tc_call(body, N, D, BR, in_specs):
    return pl.pallas_call(
        body,
        grid=(N // BR,),
        in_specs=in_specs,
        out_specs=pl.BlockSpec((BR, D), lambda i: (i, 0)),
        out_shape=jax.ShapeDtypeStruct((N, D), jnp.float32),
    )


def kernel(features, edge_index, W1, b1, W2, b2, W3, b3):
    N, D = features.shape
    E = edge_index.shape[1]
    CHA = E // (NW * KA)
    EPWA = E // NW
    BR = 2000

    EPW = E // NW
    srcd = edge_index[0].reshape(NW, EPW // 80, 80)
    dstd = edge_index[1].reshape(NW, EPW // 80, 80)
    srca = edge_index[0]
    dsta = edge_index[1]
    zND = jnp.zeros((N, D), jnp.float32)
    zN = jnp.zeros((N,), jnp.float32)
    lane = jnp.arange(D) < (D // 2)
    sel_s = jnp.broadcast_to(jnp.where(lane, 1.0, 0.0), (NW, D))
    sel_s = sel_s.astype(jnp.float32)
    sel_d = jnp.broadcast_to(jnp.where(lane, 0.0, 1.0), (NW, D))
    sel_d = sel_d.astype(jnp.float32)
    b1b = jnp.broadcast_to(b1.reshape(1, D), (8, D))
    b2b = jnp.broadcast_to(b2.reshape(1, D), (8, D))
    b3b = jnp.broadcast_to(b3.reshape(1, D), (8, D))

    hist_s, hist_d = _make_deg(N, D, EPW)(srcd, dstd, zN)
    cnt = pl.pallas_call(
        _dred_body,
        grid=(1,),
        in_specs=[pl.BlockSpec((NW, N), lambda i: (0, 0)),
                  pl.BlockSpec((NW, N), lambda i: (0, 0)),
                  pl.BlockSpec((NW, D), lambda i: (0, 0)),
                  pl.BlockSpec((NW, D), lambda i: (0, 0))],
        out_specs=pl.BlockSpec((N, D), lambda i: (0, 0)),
        out_shape=jax.ShapeDtypeStruct((N, D), jnp.float32),
    )(hist_s.reshape(NW, N), hist_d.reshape(NW, N), sel_s, sel_d)
    agg_fn = _make_agg(N, D, CHA // 1, KA, EPWA)

    spec_c = pl.BlockSpec((BR, D), lambda i: (i, 0))
    spec_a = pl.BlockSpec((NC, BR, D), lambda i: (0, i, 0))
    spec_w = pl.BlockSpec((D, D), lambda i: (0, 0))
    spec_b = pl.BlockSpec((8, D), lambda i: (0, 0))
    spec_f = pl.BlockSpec((BR, D), lambda i: (i, 0))

    hw = _tc_call(_stage0_body, N, D, BR, [spec_f, spec_c, spec_w])(
        features, cnt, W1)
    for W, bb in ((W2, b1b), (W3, b2b)):
        a = agg_fn(hw, srca, dsta, zND)
        hw = _tc_call(_stage_mid_body, N, D, BR,
                      [spec_a, spec_c, spec_b, spec_w])(a, cnt, bb, W)
    a = agg_fn(hw, srca, dsta, zND)
    out = _tc_call(_stage_last_body, N, D, BR,
                   [spec_a, spec_c, spec_b])(a, cnt, b3b)
    return out
